# baseline (speedup 1.0000x reference)
import jax
import jax.numpy as jnp
from jax import lax
from jax.experimental import pallas as pl
from jax.experimental.pallas import tpu as pltpu
from jax.experimental.pallas import tpu_sc as plsc

_NC, _NS = 2, 16
_NW = _NC * _NS


def _make_body(L, D, spw):
    def body(xc_hbm, tbl_hbm, out_hbm, xv0, xv1, buf0, buf1,
             sin0, sin1, sout0, sout1):
        xvs = (xv0, xv1)
        bufs = (buf0, buf1)
        sins = (sin0, sin1)
        souts = (sout0, sout1)
        wid = lax.axis_index("s") * _NC + lax.axis_index("c")
        base = wid * spw

        def in_copy(m, j):
            return pltpu.make_async_copy(xc_hbm.at[base + j], xvs[m], sins[m])

        def out_copy(h, j):
            return pltpu.make_async_copy(
                bufs[h], out_hbm.at[2 * (base + j) + h], souts[h])

        def assemble(m, h):
            def rowpair(i, carry):
                for k in range(4):
                    bufs[h][2 * i, pl.ds(16 * k, 16)] = \
                        xvs[m][100 * h + i, pl.ds(16 * k, 16)]
                for k in range(4):
                    bufs[h][2 * i + 1, pl.ds(16 * k, 16)] = \
                        xvs[m][100 * h + i, pl.ds(D + 16 * k, 16)]
                return carry
            lax.fori_loop(0, L // 2, rowpair, 0)

        for h in range(2):
            pltpu.sync_copy(tbl_hbm, bufs[h])
        for m in range(2):
            in_copy(m, m).start()

        def slabs(c, carry):
            for m in range(2):
                j = 2 * c + m
                in_copy(m, j).wait()
                for h in range(2):
                    @pl.when(j > 0)
                    def _():
                        out_copy(h, j - 1).wait()
                    assemble(m, h)
                    out_copy(h, j).start()

                @pl.when(j + 2 < spw)
                def _():
                    in_copy(m, j + 2).start()
            return carry

        lax.fori_loop(0, spw // 2, slabs, 0)
        for h in range(2):
            out_copy(h, spw - 1).wait()

    return body


def kernel(x, position_table):
    B, L, D = x.shape
    spw = (B // 2) // _NW
    tbl = jnp.concatenate(
        [jnp.zeros((L, D), x.dtype), position_table[:L]], axis=-1)
    packed = x.reshape(B // 2, L, 2 * D)
    mesh = plsc.VectorSubcoreMesh(core_axis_name="c", subcore_axis_name="s")
    f = pl.kernel(
        _make_body(L, D, spw),
        out_type=jax.ShapeDtypeStruct((B, L, 2 * D), x.dtype),
        mesh=mesh,
        scratch_types=(
            [pltpu.VMEM((L, 2 * D), x.dtype) for _ in range(4)]
            + [pltpu.SemaphoreType.DMA for _ in range(4)]
        ),
        compiler_params=pltpu.CompilerParams(use_tc_tiling_on_sc=True),
    )
    return f(packed, tbl)
